# Initial kernel scaffold; baseline (speedup 1.0000x reference)
import jax
import jax.numpy as jnp
from jax.experimental import pallas as pl


def kernel(x, edge_index, batch, W_init, b_init, W1, b1, W2, b2, W_out, b_out):
    raise NotImplementedError("write your pallas kernel here")


# speedup vs baseline: 12.9675x; 12.9675x over previous
import functools

import jax
import jax.numpy as jnp
from jax import lax
from jax.experimental import pallas as pl
from jax.experimental.pallas import tpu as pltpu
from jax.experimental.pallas import tpu_sc as plsc

N_NODES = 10000
N_EDGES = 320000
D = 128
N_GRAPHS = 64

NC = 2
NS = 16
K = 80
EPW = N_EDGES // (NC * NS)
CHUNKS = EPW // K
N_PAD = 10240
RPS = N_PAD // NS

_mesh = plsc.VectorSubcoreMesh(core_axis_name="c", subcore_axis_name="s")


def _sc_degree(dst, onehot, zrows):
    @functools.partial(
        pl.kernel,
        out_type=jax.ShapeDtypeStruct((NC, N_PAD, 16), jnp.float32),
        mesh=_mesh,
        scratch_types=[
            pltpu.VMEM((K,), jnp.int32),
            pltpu.VMEM((K, 16), jnp.float32),
            pltpu.VMEM_SHARED((N_PAD, 16), jnp.float32),
        ],
    )
    def k(dst_hbm, oh_hbm, z_hbm, out_hbm, idx_v, ones_v, acc):
        cid = lax.axis_index("c")
        sid = lax.axis_index("s")
        pltpu.sync_copy(oh_hbm, ones_v)
        pltpu.sync_copy(z_hbm, acc.at[pl.ds(sid * RPS, RPS)])
        plsc.subcore_barrier()
        ebase = cid * (NS * EPW) + sid * EPW

        @pl.loop(0, CHUNKS)
        def _(c):
            pltpu.sync_copy(dst_hbm.at[pl.ds(ebase + c * K, K)], idx_v)
            pltpu.sync_copy(ones_v, acc.at[idx_v], add=True)

        plsc.subcore_barrier()
        pltpu.sync_copy(acc.at[pl.ds(sid * RPS, RPS)],
                        out_hbm.at[cid, pl.ds(sid * RPS, RPS)])

    return k(dst, onehot, zrows)


def _sc_aggregate(t, src, dst, zrows):
    @functools.partial(
        pl.kernel,
        out_type=jax.ShapeDtypeStruct((NC, N_PAD, D), jnp.float32),
        mesh=_mesh,
        scratch_types=[
            pltpu.VMEM((K,), jnp.int32),
            pltpu.VMEM((K,), jnp.int32),
            pltpu.VMEM((K, D), jnp.float32),
            pltpu.VMEM_SHARED((N_PAD, D), jnp.float32),
            pltpu.SemaphoreType.DMA,
        ],
    )
    def k(t_hbm, src_hbm, dst_hbm, z_hbm, out_hbm,
          isrc_v, idst_v, rows_v, acc, sem):
        cid = lax.axis_index("c")
        sid = lax.axis_index("s")
        pltpu.sync_copy(z_hbm, acc.at[pl.ds(sid * RPS, RPS)])
        plsc.subcore_barrier()
        ebase = cid * (NS * EPW) + sid * EPW

        @pl.loop(0, CHUNKS)
        def _(c):
            pltpu.sync_copy(src_hbm.at[pl.ds(ebase + c * K, K)], isrc_v)
            pltpu.sync_copy(dst_hbm.at[pl.ds(ebase + c * K, K)], idst_v)
            pltpu.async_copy(t_hbm.at[isrc_v], rows_v, sem).wait()
            pltpu.sync_copy(rows_v, acc.at[idst_v], add=True)

        plsc.subcore_barrier()
        pltpu.sync_copy(acc.at[pl.ds(sid * RPS, RPS)],
                        out_hbm.at[cid, pl.ds(sid * RPS, RPS)])

    return k(t, src, dst, zrows)


def _dinv(dp):
    deg = 1.0 + dp[0, :N_NODES, 0:1] + dp[1, :N_NODES, 0:1]
    return lax.rsqrt(deg)


def _tc_pre(x, W_init, b_init, W1):
    def body(x_ref, wi_ref, bi_ref, w1_ref, o_ref):
        h = jnp.maximum(
            jnp.dot(x_ref[...], wi_ref[...],
                    preferred_element_type=jnp.float32) + bi_ref[...], 0.0)
        o_ref[...] = jnp.dot(h, w1_ref[...],
                             preferred_element_type=jnp.float32)

    return pl.pallas_call(
        body,
        out_shape=jax.ShapeDtypeStruct((N_NODES, D), jnp.float32),
    )(x, W_init, b_init.reshape(1, D), W1)


def _tc_scale(t, dp):
    def body(t_ref, dp_ref, o_ref):
        o_ref[...] = t_ref[...] * _dinv(dp_ref)

    return pl.pallas_call(
        body,
        out_shape=jax.ShapeDtypeStruct((N_NODES, D), jnp.float32),
    )(t, dp)


def _tc_mid(p, t, dp, b, W):
    def body(p_ref, t_ref, dp_ref, b_ref, w_ref, o_ref):
        dinv = _dinv(dp_ref)
        agg = p_ref[0, :N_NODES, :] + p_ref[1, :N_NODES, :] + t_ref[...]
        h = jnp.maximum(agg * dinv + b_ref[...], 0.0)
        o_ref[...] = jnp.dot(h, w_ref[...],
                             preferred_element_type=jnp.float32) * dinv

    return pl.pallas_call(
        body,
        out_shape=jax.ShapeDtypeStruct((N_NODES, D), jnp.float32),
    )(p, t, dp, b.reshape(1, D), W)


def _tc_post(p, t, dp, b, batch_row, W_out, b_out):
    def body(p_ref, t_ref, dp_ref, b_ref, batch_ref, wo_ref, bo_ref, o_ref):
        dinv = _dinv(dp_ref)
        agg = p_ref[0, :N_NODES, :] + p_ref[1, :N_NODES, :] + t_ref[...]
        h = jnp.maximum(agg * dinv + b_ref[...], 0.0)
        gid = lax.broadcasted_iota(jnp.int32, (N_GRAPHS, N_NODES), 0)
        maskT = jnp.where(batch_ref[...] == gid, 1.0, 0.0)
        sums = jnp.dot(maskT, h, preferred_element_type=jnp.float32)
        ones = jnp.ones((N_NODES, 8), jnp.float32)
        counts = jnp.dot(maskT, ones,
                         preferred_element_type=jnp.float32)[:, 0:1]
        g = sums / jnp.maximum(counts, 1.0)
        o_ref[...] = jnp.dot(g, wo_ref[...],
                             preferred_element_type=jnp.float32) + bo_ref[...]

    return pl.pallas_call(
        body,
        out_shape=jax.ShapeDtypeStruct((N_GRAPHS, D), jnp.float32),
    )(p, t, dp, b.reshape(1, D), batch_row, W_out, b_out.reshape(1, D))


def kernel(x, edge_index, batch, W_init, b_init, W1, b1, W2, b2, W_out, b_out):
    src = edge_index[0].astype(jnp.int32)
    dst = edge_index[1].astype(jnp.int32)
    batch_row = batch.astype(jnp.int32).reshape(1, N_NODES)

    onehot = jnp.zeros((K, 16), jnp.float32).at[:, 0].set(1.0)
    z16 = jnp.zeros((RPS, 16), jnp.float32)
    z128 = jnp.zeros((RPS, D), jnp.float32)

    dp = _sc_degree(dst, onehot, z16)
    tpre = _tc_pre(x, W_init, b_init, W1)
    t1 = _tc_scale(tpre, dp)
    p1 = _sc_aggregate(t1, src, dst, z128)
    t2 = _tc_mid(p1, t1, dp, b1, W2)
    p2 = _sc_aggregate(t2, src, dst, z128)
    return _tc_post(p2, t2, dp, b2, batch_row, W_out, b_out)


# speedup vs baseline: 23.1869x; 1.7881x over previous
import functools

import jax
import jax.numpy as jnp
from jax import lax
from jax.experimental import pallas as pl
from jax.experimental.pallas import tpu as pltpu
from jax.experimental.pallas import tpu_sc as plsc

N_NODES = 10000
N_EDGES = 320000
D = 128
N_GRAPHS = 64

NC = 2
NS = 16
K = 80
EPW = N_EDGES // (NC * NS)
CHUNKS = EPW // K
N_PAD = 10240
RPS = N_PAD // NS

_mesh = plsc.VectorSubcoreMesh(core_axis_name="c", subcore_axis_name="s")


def _sc_degree(dst, onehot, zrows):
    @functools.partial(
        pl.kernel,
        out_type=jax.ShapeDtypeStruct((NC, N_PAD, D), jnp.float32),
        mesh=_mesh,
        scratch_types=[
            pltpu.VMEM((K,), jnp.int32),
            pltpu.VMEM((K, D), jnp.float32),
            pltpu.VMEM_SHARED((N_PAD, D), jnp.float32),
        ],
    )
    def k(dst_hbm, oh_hbm, z_hbm, out_hbm, idx_v, ones_v, acc):
        cid = lax.axis_index("c")
        sid = lax.axis_index("s")
        pltpu.sync_copy(oh_hbm, ones_v)
        pltpu.sync_copy(z_hbm, acc.at[pl.ds(sid * RPS, RPS)])
        plsc.subcore_barrier()
        ebase = cid * (NS * EPW) + sid * EPW

        @pl.loop(0, CHUNKS)
        def _(c):
            pltpu.sync_copy(dst_hbm.at[pl.ds(ebase + c * K, K)], idx_v)
            pltpu.sync_copy(ones_v, acc.at[idx_v], add=True)

        plsc.subcore_barrier()
        pltpu.sync_copy(acc.at[pl.ds(sid * RPS, RPS)],
                        out_hbm.at[cid, pl.ds(sid * RPS, RPS)])

    return k(dst, onehot, zrows)


def _sc_aggregate(t, src, dst3, zrows):
    @functools.partial(
        pl.kernel,
        out_type=jax.ShapeDtypeStruct((NC, N_PAD, D), jnp.float32),
        mesh=_mesh,
        scratch_types=[
            pltpu.VMEM((EPW,), jnp.int32),
            pltpu.VMEM((CHUNKS, K), jnp.int32),
            pltpu.VMEM((K,), jnp.int32),
            pltpu.VMEM((K, D), jnp.float32),
            pltpu.VMEM((K, D), jnp.float32),
            pltpu.VMEM_SHARED((N_PAD, D), jnp.float32),
            pltpu.SemaphoreType.DMA,
            pltpu.SemaphoreType.DMA,
        ],
    )
    def k(t_hbm, src_hbm, dst_hbm, z_hbm, out_hbm,
          src_v, dst_v, idst, rows_a, rows_b, acc, sem_a, sem_b):
        cid = lax.axis_index("c")
        sid = lax.axis_index("s")
        w = cid * NS + sid
        pltpu.sync_copy(src_hbm.at[pl.ds(w * EPW, EPW)], src_v)
        pltpu.sync_copy(dst_hbm.at[w], dst_v)

        def g_start(c, buf, sem):
            pltpu.async_copy(t_hbm.at[src_v.at[pl.ds(c * K, K)]], buf, sem)

        def g_wait(c, buf, sem):
            pltpu.make_async_copy(
                t_hbm.at[src_v.at[pl.ds(c * K, K)]], buf, sem).wait()

        def scat(c, buf):
            for j in range(K // 16):
                idst[pl.ds(j * 16, 16)] = dst_v[c, pl.ds(j * 16, 16)]
            pltpu.sync_copy(buf, acc.at[idst], add=True)

        g_start(0, rows_a, sem_a)
        pltpu.sync_copy(z_hbm, acc.at[pl.ds(sid * RPS, RPS)])
        plsc.subcore_barrier()

        @pl.loop(0, CHUNKS // 2)
        def _(i):
            c0 = 2 * i
            g_start(c0 + 1, rows_b, sem_b)
            g_wait(c0, rows_a, sem_a)
            scat(c0, rows_a)
            g_start(c0 + 2, rows_a, sem_a)
            g_wait(c0 + 1, rows_b, sem_b)
            scat(c0 + 1, rows_b)

        g_wait(CHUNKS - 1, rows_a, sem_a)
        scat(CHUNKS - 1, rows_a)
        plsc.subcore_barrier()
        pltpu.sync_copy(acc.at[pl.ds(sid * RPS, RPS)],
                        out_hbm.at[cid, pl.ds(sid * RPS, RPS)])

    return k(t, src, dst3, zrows)


def _dinv(dp):
    deg = 1.0 + dp[0, :N_NODES, 0:1] + dp[1, :N_NODES, 0:1]
    return lax.rsqrt(deg)


def _tc_pre(x, W_init, b_init, W1):
    def body(x_ref, wi_ref, bi_ref, w1_ref, o_ref):
        h = jnp.maximum(
            jnp.dot(x_ref[...], wi_ref[...],
                    preferred_element_type=jnp.float32) + bi_ref[...], 0.0)
        o_ref[...] = jnp.dot(h, w1_ref[...],
                             preferred_element_type=jnp.float32)

    return pl.pallas_call(
        body,
        out_shape=jax.ShapeDtypeStruct((N_NODES, D), jnp.float32),
    )(x, W_init, b_init.reshape(1, D), W1)


def _tc_scale(t, dp):
    def body(t_ref, dp_ref, o_ref):
        o_ref[...] = t_ref[...] * _dinv(dp_ref)

    return pl.pallas_call(
        body,
        out_shape=jax.ShapeDtypeStruct((N_NODES, D), jnp.float32),
    )(t, dp)


def _tc_mid(p, t, dp, b, W):
    def body(p_ref, t_ref, dp_ref, b_ref, w_ref, o_ref):
        dinv = _dinv(dp_ref)
        agg = p_ref[0, :N_NODES, :] + p_ref[1, :N_NODES, :] + t_ref[...]
        h = jnp.maximum(agg * dinv + b_ref[...], 0.0)
        o_ref[...] = jnp.dot(h, w_ref[...],
                             preferred_element_type=jnp.float32) * dinv

    return pl.pallas_call(
        body,
        out_shape=jax.ShapeDtypeStruct((N_NODES, D), jnp.float32),
    )(p, t, dp, b.reshape(1, D), W)


def _tc_post(p, t, dp, b, batch_row, W_out, b_out):
    def body(p_ref, t_ref, dp_ref, b_ref, batch_ref, wo_ref, bo_ref, o_ref):
        dinv = _dinv(dp_ref)
        agg = p_ref[0, :N_NODES, :] + p_ref[1, :N_NODES, :] + t_ref[...]
        h = jnp.maximum(agg * dinv + b_ref[...], 0.0)
        gid = lax.broadcasted_iota(jnp.int32, (N_GRAPHS, N_NODES), 0)
        maskT = jnp.where(batch_ref[...] == gid, 1.0, 0.0)
        sums = jnp.dot(maskT, h, preferred_element_type=jnp.float32)
        ones = jnp.ones((N_NODES, 8), jnp.float32)
        counts = jnp.dot(maskT, ones,
                         preferred_element_type=jnp.float32)[:, 0:1]
        g = sums / jnp.maximum(counts, 1.0)
        o_ref[...] = jnp.dot(g, wo_ref[...],
                             preferred_element_type=jnp.float32) + bo_ref[...]

    return pl.pallas_call(
        body,
        out_shape=jax.ShapeDtypeStruct((N_GRAPHS, D), jnp.float32),
    )(p, t, dp, b.reshape(1, D), batch_row, W_out, b_out.reshape(1, D))


def kernel(x, edge_index, batch, W_init, b_init, W1, b1, W2, b2, W_out, b_out):
    src = edge_index[0].astype(jnp.int32)
    dst = edge_index[1].astype(jnp.int32)
    batch_row = batch.astype(jnp.int32).reshape(1, N_NODES)

    onehot = jnp.zeros((K, D), jnp.float32).at[:, 0].set(1.0)
    z128 = jnp.zeros((RPS, D), jnp.float32)

    dst3 = dst.reshape(NC * NS, CHUNKS, K)

    dp = _sc_degree(dst, onehot, z128)
    tpre = _tc_pre(x, W_init, b_init, W1)
    t1 = _tc_scale(tpre, dp)
    p1 = _sc_aggregate(t1, src, dst3, z128)
    t2 = _tc_mid(p1, t1, dp, b1, W2)
    p2 = _sc_aggregate(t2, src, dst3, z128)
    return _tc_post(p2, t2, dp, b2, batch_row, W_out, b_out)


# speedup vs baseline: 29.4588x; 1.2705x over previous
import functools

import jax
import jax.numpy as jnp
from jax import lax
from jax.experimental import pallas as pl
from jax.experimental.pallas import tpu as pltpu
from jax.experimental.pallas import tpu_sc as plsc

N_NODES = 10000
N_EDGES = 320000
D = 128
N_GRAPHS = 64

NC = 2
NS = 16
K = 80
EPW = N_EDGES // (NC * NS)
CHUNKS = EPW // K
N_PAD = 10240
RPS = N_PAD // NS

_mesh = plsc.VectorSubcoreMesh(core_axis_name="c", subcore_axis_name="s")


def _sc_degree(dst3, onehot, zrows):
    NB = 5

    @functools.partial(
        pl.kernel,
        out_type=jax.ShapeDtypeStruct((NC, N_PAD, D), jnp.float32),
        mesh=_mesh,
        scratch_types=(
            [pltpu.VMEM((CHUNKS, K), jnp.int32),
             pltpu.VMEM((K, D), jnp.float32),
             pltpu.VMEM_SHARED((N_PAD, D), jnp.float32)]
            + [pltpu.SemaphoreType.DMA] * NB
        ),
    )
    def k(dst_hbm, oh_hbm, z_hbm, out_hbm, dst_v, ones_v, acc, *ssems):
        cid = lax.axis_index("c")
        sid = lax.axis_index("s")
        w = cid * NS + sid
        pltpu.sync_copy(oh_hbm, ones_v)
        pltpu.sync_copy(dst_hbm.at[w], dst_v)
        pltpu.sync_copy(z_hbm, acc.at[pl.ds(sid * RPS, RPS)])
        plsc.subcore_barrier()

        def s_wait(c, b):
            pltpu.make_async_copy(ones_v, acc.at[dst_v.at[c]],
                                  ssems[b]).wait()

        @pl.loop(0, CHUNKS // NB)
        def _(i):
            for b in range(NB):
                c = i * NB + b

                @pl.when(c >= NB)
                def _():
                    s_wait(c - NB, b)

                pltpu.async_copy(ones_v, acc.at[dst_v.at[c]], ssems[b],
                                 add=True)

        for b in range(NB):
            s_wait(CHUNKS - NB + b, b)
        plsc.subcore_barrier()
        pltpu.sync_copy(acc.at[pl.ds(sid * RPS, RPS)],
                        out_hbm.at[cid, pl.ds(sid * RPS, RPS)])

    return k(dst3, onehot, zrows)


def _sc_aggregate(t, src, dst3, zrows):
    NB = 3

    @functools.partial(
        pl.kernel,
        out_type=jax.ShapeDtypeStruct((NC, N_PAD, D), jnp.float32),
        mesh=_mesh,
        scratch_types=(
            [pltpu.VMEM((CHUNKS, K), jnp.int32)]
            + [pltpu.VMEM((K,), jnp.int32)] * NB
            + [pltpu.VMEM((K, D), jnp.float32)] * NB
            + [pltpu.VMEM_SHARED((N_PAD, D), jnp.float32)]
            + [pltpu.SemaphoreType.DMA] * (3 * NB)
        ),
    )
    def k(t_hbm, src_hbm, dst_hbm, z_hbm, out_hbm, dst_v, *rest):
        isrc = rest[:NB]
        rows = rest[NB:2 * NB]
        acc = rest[2 * NB]
        isems = rest[2 * NB + 1:2 * NB + 1 + NB]
        gsems = rest[2 * NB + 1 + NB:2 * NB + 1 + 2 * NB]
        ssems = rest[2 * NB + 1 + 2 * NB:]
        cid = lax.axis_index("c")
        sid = lax.axis_index("s")
        w = cid * NS + sid
        ebase = w * EPW
        pltpu.sync_copy(dst_hbm.at[w], dst_v)

        def i_start(c, b):
            pltpu.async_copy(src_hbm.at[pl.ds(ebase + c * K, K)], isrc[b],
                             isems[b])

        def i_wait(c, b):
            pltpu.make_async_copy(src_hbm.at[pl.ds(ebase + c * K, K)],
                                  isrc[b], isems[b]).wait()

        def g_start(b):
            pltpu.async_copy(t_hbm.at[isrc[b]], rows[b], gsems[b])

        def g_wait(b):
            pltpu.make_async_copy(t_hbm.at[isrc[b]], rows[b],
                                  gsems[b]).wait()

        def s_start(c, b):
            pltpu.async_copy(rows[b], acc.at[dst_v.at[c]], ssems[b],
                             add=True)

        def s_wait(c, b):
            pltpu.make_async_copy(rows[b], acc.at[dst_v.at[c]],
                                  ssems[b]).wait()

        i_start(0, 0)
        i_start(1, 1)
        i_wait(0, 0)
        g_start(0)
        pltpu.sync_copy(z_hbm, acc.at[pl.ds(sid * RPS, RPS)])
        plsc.subcore_barrier()

        def step(c, b, cond):
            b1 = (c + 1) % NB if isinstance(c, int) else (b + 1) % NB
            b2 = (c + 2) % NB if isinstance(c, int) else (b + 2) % NB
            cond(c + 2 <= CHUNKS - 1, lambda: i_start(c + 2, b2))
            cond(c >= 2, lambda: s_wait(c - 2, b1))
            cond(c + 1 <= CHUNKS - 1,
                 lambda: (i_wait(c + 1, b1), g_start(b1)))
            g_wait(b)
            s_start(c, b)

        def t_cond(pred, fn):
            pl.when(pred)(lambda: (fn(), None)[1])

        def p_cond(pred, fn):
            if pred:
                fn()

        @pl.loop(0, (CHUNKS - 2) // NB)
        def _(i):
            for b in range(NB):
                step(i * NB + b, b, t_cond)

        for c in range(CHUNKS - 2, CHUNKS):
            step(c, c % NB, p_cond)
        for c in range(CHUNKS - 2, CHUNKS):
            s_wait(c, c % NB)
        plsc.subcore_barrier()
        pltpu.sync_copy(acc.at[pl.ds(sid * RPS, RPS)],
                        out_hbm.at[cid, pl.ds(sid * RPS, RPS)])

    return k(t, src, dst3, zrows)


def _dinv(dp):
    deg = 1.0 + dp[0, :N_NODES, 0:1] + dp[1, :N_NODES, 0:1]
    return lax.rsqrt(deg)


def _tc_pre(x, W_init, b_init, W1):
    def body(x_ref, wi_ref, bi_ref, w1_ref, o_ref):
        h = jnp.maximum(
            jnp.dot(x_ref[...], wi_ref[...],
                    preferred_element_type=jnp.float32) + bi_ref[...], 0.0)
        o_ref[...] = jnp.dot(h, w1_ref[...],
                             preferred_element_type=jnp.float32)

    return pl.pallas_call(
        body,
        out_shape=jax.ShapeDtypeStruct((N_NODES, D), jnp.float32),
    )(x, W_init, b_init.reshape(1, D), W1)


def _tc_scale(t, dp):
    def body(t_ref, dp_ref, o_ref):
        o_ref[...] = t_ref[...] * _dinv(dp_ref)

    return pl.pallas_call(
        body,
        out_shape=jax.ShapeDtypeStruct((N_NODES, D), jnp.float32),
    )(t, dp)


def _tc_mid(p, t, dp, b, W):
    def body(p_ref, t_ref, dp_ref, b_ref, w_ref, o_ref):
        dinv = _dinv(dp_ref)
        agg = p_ref[0, :N_NODES, :] + p_ref[1, :N_NODES, :] + t_ref[...]
        h = jnp.maximum(agg * dinv + b_ref[...], 0.0)
        o_ref[...] = jnp.dot(h, w_ref[...],
                             preferred_element_type=jnp.float32) * dinv

    return pl.pallas_call(
        body,
        out_shape=jax.ShapeDtypeStruct((N_NODES, D), jnp.float32),
    )(p, t, dp, b.reshape(1, D), W)


def _tc_post(p, t, dp, b, batch_row, W_out, b_out):
    def body(p_ref, t_ref, dp_ref, b_ref, batch_ref, wo_ref, bo_ref, o_ref):
        dinv = _dinv(dp_ref)
        agg = p_ref[0, :N_NODES, :] + p_ref[1, :N_NODES, :] + t_ref[...]
        h = jnp.maximum(agg * dinv + b_ref[...], 0.0)
        gid = lax.broadcasted_iota(jnp.int32, (N_GRAPHS, N_NODES), 0)
        maskT = jnp.where(batch_ref[...] == gid, 1.0, 0.0)
        sums = jnp.dot(maskT, h, preferred_element_type=jnp.float32)
        ones = jnp.ones((N_NODES, 8), jnp.float32)
        counts = jnp.dot(maskT, ones,
                         preferred_element_type=jnp.float32)[:, 0:1]
        g = sums / jnp.maximum(counts, 1.0)
        o_ref[...] = jnp.dot(g, wo_ref[...],
                             preferred_element_type=jnp.float32) + bo_ref[...]

    return pl.pallas_call(
        body,
        out_shape=jax.ShapeDtypeStruct((N_GRAPHS, D), jnp.float32),
    )(p, t, dp, b.reshape(1, D), batch_row, W_out, b_out.reshape(1, D))


def kernel(x, edge_index, batch, W_init, b_init, W1, b1, W2, b2, W_out, b_out):
    src = edge_index[0].astype(jnp.int32)
    dst = edge_index[1].astype(jnp.int32)
    batch_row = batch.astype(jnp.int32).reshape(1, N_NODES)

    onehot = jnp.zeros((K, D), jnp.float32).at[:, 0].set(1.0)
    z128 = jnp.zeros((RPS, D), jnp.float32)

    dst3 = dst.reshape(NC * NS, CHUNKS, K)

    dp = _sc_degree(dst3, onehot, z128)
    tpre = _tc_pre(x, W_init, b_init, W1)
    t1 = _tc_scale(tpre, dp)
    p1 = _sc_aggregate(t1, src, dst3, z128)
    t2 = _tc_mid(p1, t1, dp, b1, W2)
    p2 = _sc_aggregate(t2, src, dst3, z128)
    return _tc_post(p2, t2, dp, b2, batch_row, W_out, b_out)
